# baseline (speedup 1.0000x reference)
import jax
import jax.numpy as jnp
import numpy as np
from jax.experimental import pallas as pl

_K_KNN = 12
_K_SS = 4
_BB = 16


def _body(single_ref, pair0_ref, coords_ref, coordsT_ref, css_ref,
          ln_w_ref, ln_b_ref, wqT_ref, wkT_ref, wvT_ref, webT_ref,
          woT_ref, bo_ref, eye_ref, selmask_ref, pool_ref, out_ref):
    f32 = jnp.float32
    BB, L, D = single_ref.shape
    H = webT_ref.shape[1]
    Dh = D // H
    E = _K_KNN + _K_SS
    R = BB * L
    scale = f32(Dh ** -0.5)

    x2 = single_ref[...].reshape(R, D)
    mu = jnp.mean(x2, axis=1, keepdims=True)
    xc = x2 - mu
    var = jnp.mean(xc * xc, axis=1, keepdims=True)
    xn = xc / jnp.sqrt(var + 1e-5) * ln_w_ref[...] + ln_b_ref[...]

    q = jnp.dot(xn, wqT_ref[...], preferred_element_type=f32)
    k = jnp.dot(xn, wkT_ref[...], preferred_element_type=f32)
    v = jnp.dot(xn, wvT_ref[...], preferred_element_type=f32)

    lanesR = jax.lax.broadcasted_iota(jnp.int32, (R, L), 1)

    s2 = None
    for a in range(3):
        colv = coords_ref[...].reshape(R, 3)[:, a:a + 1]
        rowv = jnp.broadcast_to(coordsT_ref[:, a:a + 1, :],
                                (BB, L, L)).reshape(R, L)
        dc = colv - rowv
        s2 = dc * dc if s2 is None else s2 + dc * dc
    negd = -(jnp.sqrt(s2) + eye_ref[...])
    css2 = css_ref[...].reshape(R, L)

    neg_inf = f32(-jnp.inf)

    def topk_cols(vals, kk):
        cols = []
        vc = vals
        for _ in range(kk):
            am = jnp.argmax(vc, axis=1, keepdims=True).astype(jnp.int32)
            cols.append(am)
            vc = jnp.where(lanesR == am, neg_inf, vc)
        return cols

    both = jnp.concatenate([negd, css2], axis=0)
    lanes2R = jax.lax.broadcasted_iota(jnp.int32, (2 * R, L), 1)
    vc2 = both
    cols2 = []
    for _ in range(_K_SS):
        am = jnp.argmax(vc2, axis=1, keepdims=True).astype(jnp.int32)
        cols2.append(am)
        vc2 = jnp.where(lanes2R == am, neg_inf, vc2)
    knn_cols = [c[:R] for c in cols2] + topk_cols(vc2[:R], _K_KNN - _K_SS)
    ss_cols = [c[R:] for c in cols2]
    cols = knn_cols + ss_cols

    lanesEL = jax.lax.broadcasted_iota(jnp.int32, (E * L, L), 1)

    G = 4
    ctx_parts = [None] * BB
    for g0 in range(0, BB, G):
        bbs = list(range(g0, g0 + G))
        catcols = [jnp.concatenate(
            [cols[e][bb * L:(bb + 1) * L] for e in range(E)], axis=0)
            for bb in bbs]
        mstacks = [(cc == lanesEL).astype(f32) for cc in catcols]
        r_alls = [jnp.dot(mstacks[j], pair0_ref[bb, 0],
                          preferred_element_type=f32)
                  for j, bb in enumerate(bbs)]
        bmats = [jnp.dot(ra * selmask_ref[...], webT_ref[...],
                         preferred_element_type=f32) for ra in r_alls]
        bias_ts = [jnp.dot(pool_ref[...], bm, preferred_element_type=f32)
                   for bm in bmats]
        bmaxs = [jnp.max(bt, axis=0, keepdims=True) for bt in bias_ts]
        expbs = [jnp.exp(bias_ts[j] - bmaxs[j]) for j in range(G)]

        bhs = [(j, bb, h) for j, bb in enumerate(bbs) for h in range(H)]
        shs = [jax.lax.dot_general(
            q[bb * L:(bb + 1) * L, h * Dh:(h + 1) * Dh],
            k[bb * L:(bb + 1) * L, h * Dh:(h + 1) * Dh],
            (((1,), (1,)), ((), ())), preferred_element_type=f32)
            for (j, bb, h) in bhs]
        sscal = [sh * scale for sh in shs]
        ms = [jnp.max(t, axis=1, keepdims=True) for t in sscal]
        es = [jnp.exp(sscal[i] - ms[i]) for i in range(len(bhs))]
        gs = [None] * len(bhs)
        for j in range(G):
            for e in range(E):
                me = mstacks[j][e * L:(e + 1) * L, :]
                for h in range(H):
                    i = j * H + h
                    w = expbs[j][e:e + 1, h:h + 1]
                    gs[i] = me * w if gs[i] is None else gs[i] + me * w
        ps = [es[i] * gs[i] for i in range(len(bhs))]
        sums = [jnp.sum(p, axis=1, keepdims=True) for p in ps]
        attns = [ps[i] / sums[i] for i in range(len(bhs))]
        ctxs = [jnp.dot(attns[i], v[bb * L:(bb + 1) * L, h * Dh:(h + 1) * Dh],
                        preferred_element_type=f32)
                for i, (j, bb, h) in enumerate(bhs)]
        for j, bb in enumerate(bbs):
            ctx_parts[bb] = jnp.concatenate(ctxs[j * H:(j + 1) * H], axis=1)
    ctx = jnp.concatenate(ctx_parts, axis=0)
    out2 = x2 + jnp.dot(ctx, woT_ref[...],
                        preferred_element_type=f32) + bo_ref[...]
    out_ref[...] = out2.reshape(BB, L, D)


def kernel(single, pair, coords, contact_ss, ln_w, ln_b, Wq, Wk, Wv, W_eb,
           Wo, bo):
    B, L, D = single.shape
    DP = pair.shape[-1]
    BB = _BB
    E = _K_KNN + _K_SS
    R = BB * L
    coordsT = jnp.transpose(coords, (0, 2, 1))
    ii = np.arange(R)[:, None] % L
    jj = np.arange(L)[None, :]
    eye_big = jnp.asarray(np.where(ii == jj, 1e9, 0.0).astype(np.float32))
    rEL = np.arange(E * L)[:, None]
    selmask = jnp.asarray(
        (jj == (16 * ((rEL % L) % 4) + rEL // L)).astype(np.float32))
    rp = np.arange(E * L)[None, :]
    ep = np.arange(E)[:, None]
    pool = jnp.asarray((((rp % L) // 4) == ep).astype(np.float32))
    args = (single, pair, coords, coordsT, contact_ss,
            ln_w.reshape(1, D), ln_b.reshape(1, D),
            Wq.T, Wk.T, Wv.T, W_eb.T, Wo.T, bo.reshape(1, D),
            eye_big, selmask, pool)
    in_specs = [
        pl.BlockSpec((BB, L, D), lambda b: (b, 0, 0)),
        pl.BlockSpec((BB, 1, L, DP), lambda b: (b, 0, 0, 0)),
        pl.BlockSpec((BB, L, 3), lambda b: (b, 0, 0)),
        pl.BlockSpec((BB, 3, L), lambda b: (b, 0, 0)),
        pl.BlockSpec((BB, L, L), lambda b: (b, 0, 0)),
        pl.BlockSpec((1, D), lambda b: (0, 0)),
        pl.BlockSpec((1, D), lambda b: (0, 0)),
        pl.BlockSpec((D, D), lambda b: (0, 0)),
        pl.BlockSpec((D, D), lambda b: (0, 0)),
        pl.BlockSpec((D, D), lambda b: (0, 0)),
        pl.BlockSpec((DP, W_eb.shape[0]), lambda b: (0, 0)),
        pl.BlockSpec((D, D), lambda b: (0, 0)),
        pl.BlockSpec((1, D), lambda b: (0, 0)),
        pl.BlockSpec((R, L), lambda b: (0, 0)),
        pl.BlockSpec((E * L, L), lambda b: (0, 0)),
        pl.BlockSpec((E, E * L), lambda b: (0, 0)),
    ]
    return pl.pallas_call(
        _body,
        grid=(B // BB,),
        in_specs=in_specs,
        out_specs=pl.BlockSpec((BB, L, D), lambda b: (b, 0, 0)),
        out_shape=jax.ShapeDtypeStruct((B, L, D), jnp.float32),
    )(*args)


# speedup vs baseline: 1.0409x; 1.0409x over previous
import jax
import jax.numpy as jnp
import numpy as np
from jax.experimental import pallas as pl

_K_KNN = 12
_K_SS = 4
_BB = 16


def _body(single_ref, pair0_ref, coords_ref, coordsT_ref, css_ref,
          ln_w_ref, ln_b_ref, wqT_ref, wkT_ref, wvT_ref, webT_ref,
          woT_ref, bo_ref, eye_ref, selmask_ref, pool_ref, out_ref):
    f32 = jnp.float32
    BB, L, D = single_ref.shape
    H = webT_ref.shape[1]
    Dh = D // H
    E = _K_KNN + _K_SS
    R = BB * L
    scale = f32(Dh ** -0.5)

    x2 = single_ref[...].reshape(R, D)
    mu = jnp.mean(x2, axis=1, keepdims=True)
    xc = x2 - mu
    var = jnp.mean(xc * xc, axis=1, keepdims=True)
    xn = xc / jnp.sqrt(var + 1e-5) * ln_w_ref[...] + ln_b_ref[...]

    q = jnp.dot(xn, wqT_ref[...], preferred_element_type=f32)
    k = jnp.dot(xn, wkT_ref[...], preferred_element_type=f32)
    v = jnp.dot(xn, wvT_ref[...], preferred_element_type=f32)

    lanesR = jax.lax.broadcasted_iota(jnp.int32, (R, L), 1)

    s2 = None
    for a in range(3):
        colv = coords_ref[...].reshape(R, 3)[:, a:a + 1]
        rowv = jnp.broadcast_to(coordsT_ref[:, a:a + 1, :],
                                (BB, L, L)).reshape(R, L)
        dc = colv - rowv
        s2 = dc * dc if s2 is None else s2 + dc * dc
    negd = -(jnp.sqrt(s2) + eye_ref[...])
    css2 = css_ref[...].reshape(R, L)

    neg_inf = f32(-jnp.inf)

    def topk_cols(vals, kk):
        cols = []
        vc = vals
        for _ in range(kk):
            am = jnp.argmax(vc, axis=1, keepdims=True).astype(jnp.int32)
            cols.append(am)
            vc = jnp.where(lanesR == am, neg_inf, vc)
        return cols

    both = jnp.concatenate([negd, css2], axis=0)
    lanes2R = jax.lax.broadcasted_iota(jnp.int32, (2 * R, L), 1)
    vc2 = both
    cols2 = []
    for _ in range(_K_SS):
        am = jnp.argmax(vc2, axis=1, keepdims=True).astype(jnp.int32)
        cols2.append(am)
        vc2 = jnp.where(lanes2R == am, neg_inf, vc2)
    knn_cols = [c[:R] for c in cols2] + topk_cols(vc2[:R], _K_KNN - _K_SS)
    ss_cols = [c[R:] for c in cols2]
    cols = knn_cols + ss_cols

    lanesEL = jax.lax.broadcasted_iota(jnp.int32, (E * L, L), 1)

    catcols = [jnp.concatenate([cols[e][bb * L:(bb + 1) * L] for e in range(E)],
                               axis=0) for bb in range(BB)]
    mstacks = [(cc == lanesEL).astype(f32) for cc in catcols]
    r_alls = [jnp.dot(mstacks[bb], pair0_ref[bb, 0],
                      preferred_element_type=f32) for bb in range(BB)]
    bmats = [jnp.dot(r_alls[bb] * selmask_ref[...], webT_ref[...],
                     preferred_element_type=f32) for bb in range(BB)]
    bias_ts = [jnp.dot(pool_ref[...], bmats[bb], preferred_element_type=f32)
               for bb in range(BB)]
    bmaxs = [jnp.max(bt, axis=0, keepdims=True) for bt in bias_ts]
    expbs = [jnp.exp(bias_ts[bb] - bmaxs[bb]) for bb in range(BB)]

    bhs = [(bb, h) for bb in range(BB) for h in range(H)]
    shs = [jax.lax.dot_general(
        q[bb * L:(bb + 1) * L, h * Dh:(h + 1) * Dh],
        k[bb * L:(bb + 1) * L, h * Dh:(h + 1) * Dh],
        (((1,), (1,)), ((), ())), preferred_element_type=f32)
        for (bb, h) in bhs]
    sscal = [sh * scale for sh in shs]
    ms = [jnp.max(t, axis=1, keepdims=True) for t in sscal]
    es = [jnp.exp(sscal[i] - ms[i]) for i in range(len(bhs))]
    gs = [None] * len(bhs)
    for bb in range(BB):
        for e in range(E):
            me = mstacks[bb][e * L:(e + 1) * L, :]
            for h in range(H):
                i = bb * H + h
                w = expbs[bb][e:e + 1, h:h + 1]
                gs[i] = me * w if gs[i] is None else gs[i] + me * w
    ps = [es[i] * gs[i] for i in range(len(bhs))]
    sums = [jnp.sum(p, axis=1, keepdims=True) for p in ps]
    attns = [ps[i] / sums[i] for i in range(len(bhs))]
    ctxs = [jnp.dot(attns[i], v[bb * L:(bb + 1) * L, h * Dh:(h + 1) * Dh],
                    preferred_element_type=f32)
            for i, (bb, h) in enumerate(bhs)]
    ctx_parts = [jnp.concatenate(ctxs[bb * H:(bb + 1) * H], axis=1)
                 for bb in range(BB)]
    ctx = jnp.concatenate(ctx_parts, axis=0)
    out2 = x2 + jnp.dot(ctx, woT_ref[...],
                        preferred_element_type=f32) + bo_ref[...]
    out_ref[...] = out2.reshape(BB, L, D)


def kernel(single, pair, coords, contact_ss, ln_w, ln_b, Wq, Wk, Wv, W_eb,
           Wo, bo):
    B, L, D = single.shape
    DP = pair.shape[-1]
    BB = _BB
    E = _K_KNN + _K_SS
    R = BB * L
    coordsT = jnp.transpose(coords, (0, 2, 1))
    ii = np.arange(R)[:, None] % L
    jj = np.arange(L)[None, :]
    eye_big = jnp.asarray(np.where(ii == jj, 1e9, 0.0).astype(np.float32))
    rEL = np.arange(E * L)[:, None]
    selmask = jnp.asarray(
        (jj == (16 * ((rEL % L) % 4) + rEL // L)).astype(np.float32))
    rp = np.arange(E * L)[None, :]
    ep = np.arange(E)[:, None]
    pool = jnp.asarray((((rp % L) // 4) == ep).astype(np.float32))
    args = (single, pair, coords, coordsT, contact_ss,
            ln_w.reshape(1, D), ln_b.reshape(1, D),
            Wq.T, Wk.T, Wv.T, W_eb.T, Wo.T, bo.reshape(1, D),
            eye_big, selmask, pool)
    in_specs = [
        pl.BlockSpec((BB, L, D), lambda b: (b, 0, 0)),
        pl.BlockSpec((BB, 1, L, DP), lambda b: (b, 0, 0, 0)),
        pl.BlockSpec((BB, L, 3), lambda b: (b, 0, 0)),
        pl.BlockSpec((BB, 3, L), lambda b: (b, 0, 0)),
        pl.BlockSpec((BB, L, L), lambda b: (b, 0, 0)),
        pl.BlockSpec((1, D), lambda b: (0, 0)),
        pl.BlockSpec((1, D), lambda b: (0, 0)),
        pl.BlockSpec((D, D), lambda b: (0, 0)),
        pl.BlockSpec((D, D), lambda b: (0, 0)),
        pl.BlockSpec((D, D), lambda b: (0, 0)),
        pl.BlockSpec((DP, W_eb.shape[0]), lambda b: (0, 0)),
        pl.BlockSpec((D, D), lambda b: (0, 0)),
        pl.BlockSpec((1, D), lambda b: (0, 0)),
        pl.BlockSpec((R, L), lambda b: (0, 0)),
        pl.BlockSpec((E * L, L), lambda b: (0, 0)),
        pl.BlockSpec((E, E * L), lambda b: (0, 0)),
    ]
    return pl.pallas_call(
        _body,
        grid=(B // BB,),
        in_specs=in_specs,
        out_specs=pl.BlockSpec((BB, L, D), lambda b: (b, 0, 0)),
        out_shape=jax.ShapeDtypeStruct((B, L, D), jnp.float32),
    )(*args)
